# Initial kernel scaffold; baseline (speedup 1.0000x reference)
import jax
import jax.numpy as jnp
from jax.experimental import pallas as pl


def kernel(hidden_states, gate_w, w13, w2):
    raise NotImplementedError("write your pallas kernel here")


# speedup vs baseline: 2.1545x; 2.1545x over previous
import functools

import jax
import jax.numpy as jnp
from jax.experimental import pallas as pl
from jax.experimental.pallas import tpu as pltpu

E = 8
TOPK = 2
D = 1024
DFF = 768
T = 2048


def _router_body(x_ref, gw_ref, logits_ref, coef_ref):
    x = x_ref[...].astype(jnp.bfloat16)
    gw = gw_ref[...].astype(jnp.bfloat16)
    logits = jax.lax.dot_general(
        x, gw, (((1,), (1,)), ((), ())),
        preferred_element_type=jnp.float32,
    )
    logits_ref[...] = logits
    m = jnp.max(logits, axis=-1, keepdims=True)
    p = jnp.exp(logits - m)
    probs = p / jnp.sum(p, axis=-1, keepdims=True)
    ids = jax.lax.broadcasted_iota(jnp.int32, (T, E), 1)
    m1 = jnp.max(probs, axis=-1, keepdims=True)
    i1 = jnp.min(jnp.where(probs == m1, ids, E), axis=-1, keepdims=True)
    probs2 = jnp.where(ids == i1, -1.0, probs)
    m2 = jnp.max(probs2, axis=-1, keepdims=True)
    i2 = jnp.min(jnp.where(probs2 == m2, ids, E), axis=-1, keepdims=True)
    denom = m1 + m2
    w1 = m1 / denom
    w2 = m2 / denom
    coef_ref[...] = jnp.where(ids == i1, w1, 0.0) + jnp.where(ids == i2, w2, 0.0)


def _expert_body(x_ref, w13_ref, w2_ref, coef_ref, out_ref):
    e = pl.program_id(0)
    ids = jax.lax.broadcasted_iota(jnp.int32, (T, E), 1)
    coef = jnp.sum(jnp.where(ids == e, coef_ref[...], 0.0), axis=-1, keepdims=True)
    BT = 256
    for t in range(T // BT):
        sl = pl.ds(t * BT, BT)
        xs = x_ref[sl, :].astype(jnp.bfloat16)
        h = jax.lax.dot_general(
            xs, w13_ref[0].astype(jnp.bfloat16), (((1,), (1,)), ((), ())),
            preferred_element_type=jnp.float32,
        )
        g = h[:, :DFF]
        u = h[:, DFF:]
        a = (g / (1.0 + jnp.exp(-g))) * u
        y = jax.lax.dot_general(
            a.astype(jnp.bfloat16), w2_ref[0].astype(jnp.bfloat16),
            (((1,), (1,)), ((), ())),
            preferred_element_type=jnp.float32,
        )
        contrib = coef[t * BT:(t + 1) * BT, :] * y

        @pl.when(e == 0)
        def _():
            out_ref[sl, :] = contrib

        @pl.when(e != 0)
        def _():
            out_ref[sl, :] = out_ref[sl, :] + contrib


@functools.partial(jax.jit, static_argnames=("interpret",))
def kernel(hidden_states, gate_w, w13, w2, interpret=False):
    x = hidden_states.reshape(T, D)
    logits, coef = pl.pallas_call(
        _router_body,
        out_shape=(
            jax.ShapeDtypeStruct((T, E), jnp.float32),
            jax.ShapeDtypeStruct((T, E), jnp.float32),
        ),
        interpret=interpret,
    )(x, gate_w)
    out = pl.pallas_call(
        _expert_body,
        grid=(E,),
        in_specs=[
            pl.BlockSpec((T, D), lambda e: (0, 0)),
            pl.BlockSpec((1, 2 * DFF, D), lambda e: (e, 0, 0)),
            pl.BlockSpec((1, D, DFF), lambda e: (e, 0, 0)),
            pl.BlockSpec((T, E), lambda e: (0, 0)),
        ],
        out_specs=pl.BlockSpec((T, D), lambda e: (0, 0)),
        out_shape=jax.ShapeDtypeStruct((T, D), jnp.float32),
        interpret=interpret,
    )(x, w13, w2, coef)
    return out, logits
